# initial kernel scaffold (unmeasured)
import jax
import jax.numpy as jnp
from jax import lax
from jax.experimental import pallas as pl
from jax.experimental.pallas import tpu as pltpu

N_DEV = 32
MB = 128
KB = 128


def kernel(x, w_mat):
    m_loc, k_loc = x.shape
    k_full, n = w_mat.shape

    def body(x_ref, w_ref, out_ref, xg_ref, y_ref, amax_src_ref,
             amax_buf_ref, send_sems, recv_sems, a_send_sems, a_recv_sems):
        my = lax.axis_index("i")

        barrier = pltpu.get_barrier_semaphore()
        for dj in range(1, N_DEV):
            tgt = lax.rem(my + dj, N_DEV)
            pl.semaphore_signal(
                barrier, inc=1,
                device_id=(tgt,), device_id_type=pl.DeviceIdType.MESH,
            )
        pl.semaphore_wait(barrier, N_DEV - 1)

        xg_ref[:, pl.ds(my * KB, KB)] = x_ref[pl.ds(my * MB, MB), :]

        rdmas = []
        for dj in range(1, N_DEV):
            tgt = lax.rem(my + dj, N_DEV)
            rdma = pltpu.make_async_remote_copy(
                src_ref=x_ref.at[pl.ds(tgt * MB, MB), :],
                dst_ref=xg_ref.at[:, pl.ds(my * KB, KB)],
                send_sem=send_sems.at[dj - 1],
                recv_sem=recv_sems.at[dj - 1],
                device_id=(tgt,),
                device_id_type=pl.DeviceIdType.MESH,
            )
            rdma.start()
            rdmas.append(rdma)

        for dj in range(1, N_DEV):
            rdmas[dj - 1].wait_recv()

        y = jnp.dot(xg_ref[:, :], w_ref[:, :],
                    preferred_element_type=jnp.float32)
        y_ref[:, :] = y

        for dj in range(1, N_DEV):
            rdmas[dj - 1].wait_send()

        local_amax = jnp.max(jnp.abs(y))
        amax_src_ref[:, :] = jnp.full((8, 128), local_amax, jnp.float32)
        amax_buf_ref[pl.ds(my, 1), :, :] = amax_src_ref[:, :][None]
        a_rdmas = []
        for dj in range(1, N_DEV):
            tgt = lax.rem(my + dj, N_DEV)
            r = pltpu.make_async_remote_copy(
                src_ref=amax_src_ref,
                dst_ref=amax_buf_ref.at[my],
                send_sem=a_send_sems.at[dj - 1],
                recv_sem=a_recv_sems.at[dj - 1],
                device_id=(tgt,),
                device_id_type=pl.DeviceIdType.MESH,
            )
            r.start()
            a_rdmas.append(r)
        for dj in range(1, N_DEV):
            a_rdmas[dj - 1].wait_recv()

        gmax = jnp.max(amax_buf_ref[:, :, :])

        q = (y_ref[:, :] * (448.0 / gmax)).astype(jnp.float8_e4m3fn)
        out_ref[:, :] = q.astype(jnp.float32) * (gmax / 448.0)

        for dj in range(1, N_DEV):
            a_rdmas[dj - 1].wait_send()

    return pl.pallas_call(
        body,
        out_shape=jax.ShapeDtypeStruct((MB, n), jnp.float32),
        in_specs=[
            pl.BlockSpec(memory_space=pltpu.VMEM),
            pl.BlockSpec(memory_space=pltpu.VMEM),
        ],
        out_specs=pl.BlockSpec(memory_space=pltpu.VMEM),
        scratch_shapes=[
            pltpu.VMEM((MB, k_full), jnp.bfloat16),
            pltpu.VMEM((MB, n), jnp.float32),
            pltpu.VMEM((8, 128), jnp.float32),
            pltpu.VMEM((N_DEV, 8, 128), jnp.float32),
            pltpu.SemaphoreType.DMA((N_DEV - 1,)),
            pltpu.SemaphoreType.DMA((N_DEV - 1,)),
            pltpu.SemaphoreType.DMA((N_DEV - 1,)),
            pltpu.SemaphoreType.DMA((N_DEV - 1,)),
        ],
        compiler_params=pltpu.CompilerParams(collective_id=0),
    )(x, w_mat)


# baseline (device time: 37397 ns/iter reference)
import jax
import jax.numpy as jnp
from jax import lax
from jax.experimental import pallas as pl
from jax.experimental.pallas import tpu as pltpu

N_DEV = 32
MB = 128
KB = 128
WCHUNK = 512
NW = 4096 // WCHUNK


def kernel(x, w_mat):
    m_loc, k_loc = x.shape
    k_full, n = w_mat.shape

    def body(x_ref, w_hbm, out_ref, xb_ref, wstage_ref, wb_ref, xg_ref,
             y_ref, amax_src_ref, amax_buf_ref, send_sems, recv_sems,
             a_send_sems, a_recv_sems, w_sems):
        my = lax.axis_index("i")

        barrier = pltpu.get_barrier_semaphore()
        for dj in range(1, N_DEV):
            tgt = lax.rem(my + dj, N_DEV)
            pl.semaphore_signal(
                barrier, inc=1,
                device_id=(tgt,), device_id_type=pl.DeviceIdType.MESH,
            )
        pl.semaphore_wait(barrier, N_DEV - 1)

        xb_ref[:, :] = x_ref[:, :].astype(jnp.bfloat16)

        xg_ref[:, pl.ds(my * KB, KB)] = xb_ref[pl.ds(my * MB, MB), :]

        rdmas = []
        for dj in range(1, N_DEV):
            tgt = lax.rem(my + dj, N_DEV)
            rdma = pltpu.make_async_remote_copy(
                src_ref=xb_ref.at[pl.ds(tgt * MB, MB), :],
                dst_ref=xg_ref.at[:, pl.ds(my * KB, KB)],
                send_sem=send_sems.at[dj - 1],
                recv_sem=recv_sems.at[dj - 1],
                device_id=(tgt,),
                device_id_type=pl.DeviceIdType.MESH,
            )
            rdma.start()
            rdmas.append(rdma)

        wcopies = [
            pltpu.make_async_copy(
                w_hbm.at[pl.ds(c * WCHUNK, WCHUNK), :],
                wstage_ref.at[c % 2],
                w_sems.at[c % 2],
            )
            for c in range(NW)
        ]
        wcopies[0].start()
        for c in range(NW):
            wcopies[c].wait()
            if c + 1 < NW:
                wcopies[c + 1].start()
            wb_ref[pl.ds(c * WCHUNK, WCHUNK), :] = (
                wstage_ref[c % 2].astype(jnp.bfloat16))

        for dj in range(1, N_DEV):
            rdmas[dj - 1].wait_recv()

        y = jnp.dot(xg_ref[:, :], wb_ref[:, :],
                    preferred_element_type=jnp.float32)
        y_ref[:, :] = y

        for dj in range(1, N_DEV):
            rdmas[dj - 1].wait_send()

        local_amax = jnp.max(jnp.abs(y))
        amax_src_ref[:, :] = jnp.full((8, 128), local_amax, jnp.float32)
        amax_buf_ref[pl.ds(my, 1), :, :] = amax_src_ref[:, :][None]
        a_rdmas = []
        for dj in range(1, N_DEV):
            tgt = lax.rem(my + dj, N_DEV)
            r = pltpu.make_async_remote_copy(
                src_ref=amax_src_ref,
                dst_ref=amax_buf_ref.at[my],
                send_sem=a_send_sems.at[dj - 1],
                recv_sem=a_recv_sems.at[dj - 1],
                device_id=(tgt,),
                device_id_type=pl.DeviceIdType.MESH,
            )
            r.start()
            a_rdmas.append(r)
        for dj in range(1, N_DEV):
            a_rdmas[dj - 1].wait_recv()

        gmax = jnp.max(amax_buf_ref[:, :, :])

        q = (y_ref[:, :] * (448.0 / gmax)).astype(jnp.float8_e4m3fn)
        out_ref[:, :] = q.astype(jnp.float32) * (gmax / 448.0)

        for dj in range(1, N_DEV):
            a_rdmas[dj - 1].wait_send()

    return pl.pallas_call(
        body,
        out_shape=jax.ShapeDtypeStruct((MB, n), jnp.float32),
        in_specs=[
            pl.BlockSpec(memory_space=pltpu.VMEM),
            pl.BlockSpec(memory_space=pl.ANY),
        ],
        out_specs=pl.BlockSpec(memory_space=pltpu.VMEM),
        scratch_shapes=[
            pltpu.VMEM((m_loc, k_loc), jnp.bfloat16),
            pltpu.VMEM((2, WCHUNK, n), jnp.float32),
            pltpu.VMEM((k_full, n), jnp.bfloat16),
            pltpu.VMEM((MB, k_full), jnp.bfloat16),
            pltpu.VMEM((MB, n), jnp.float32),
            pltpu.VMEM((8, 128), jnp.float32),
            pltpu.VMEM((N_DEV, 8, 128), jnp.float32),
            pltpu.SemaphoreType.DMA((N_DEV - 1,)),
            pltpu.SemaphoreType.DMA((N_DEV - 1,)),
            pltpu.SemaphoreType.DMA((N_DEV - 1,)),
            pltpu.SemaphoreType.DMA((N_DEV - 1,)),
            pltpu.SemaphoreType.DMA((2,)),
        ],
        compiler_params=pltpu.CompilerParams(
            collective_id=0, vmem_limit_bytes=60 * 1024 * 1024),
    )(x, w_mat)
